# baseline (speedup 1.0000x reference)
import functools

import jax
import jax.numpy as jnp
from jax import lax
from jax.experimental import pallas as pl
from jax.experimental.pallas import tpu as pltpu
from jax.experimental.pallas import tpu_sc as plsc

K = 16
NB = 4
LN2 = 0.6931471805599453
SQRT2 = 1.41421356


def _table_body(uwt_ref, uht_ref, out_ref):
    uw = uwt_ref[...]
    uh = uht_ref[...]
    m = jnp.max(uw, axis=0, keepdims=True)
    e = jnp.exp(uw - m)
    w = e / jnp.sum(e, axis=0, keepdims=True)
    uhe = jnp.exp(uh)
    hlo = uhe[:K, :]
    hext = uhe[1:K + 1, :]
    pair = 0.5 * (hlo + hext) * w
    area = jnp.sum(pair, axis=0, keepdims=True)
    inv_area = 1.0 / area
    trap = pair * inv_area

    def cumsum0(a):
        for s in (1, 2, 4, 8):
            zz = jnp.zeros((s, a.shape[1]), jnp.float32)
            a = a + jnp.concatenate([zz, a[:K - s, :]], axis=0)
        return a

    cdfc = cumsum0(trap)
    locc = cumsum0(w)
    z = jnp.zeros((1, uw.shape[1]), jnp.float32)
    loc0 = jnp.concatenate([z, locc[:K - 1, :]], axis=0)
    cdf0 = jnp.concatenate([z, cdfc[:K - 1, :]], axis=0)
    h0 = hlo * inv_area
    dd = (hext - hlo) * inv_area
    ql = jnp.clip((loc0 * 65535.0 + 0.5).astype(jnp.int32), 0, 65535)
    lo = jnp.concatenate([ql[2 * j:2 * j + 1] for j in range(8)], axis=0)
    hi = jnp.concatenate([ql[2 * j + 1:2 * j + 2] for j in range(8)], axis=0)
    packed = lax.bitcast_convert_type(
        lax.bitwise_or(lo, lax.shift_left(hi, 16)), jnp.float32)
    zpad = jnp.zeros_like(w)
    bins = jnp.stack([loc0, w, 1.0 / w, cdf0, h0, dd, zpad, zpad], axis=1)
    out_ref[...] = jnp.concatenate([packed[None], bins], axis=0)


def _build_table(uw, uh):
    r = uw.shape[0]
    bl = 6400
    rp = -(-r // bl) * bl
    uwt = jnp.pad(uw.T, ((0, 0), (0, rp - r)))
    uht = jnp.pad(uh.T, ((0, 0), (0, rp - r)))
    tab_t = pl.pallas_call(
        _table_body,
        grid=(rp // bl,),
        in_specs=[
            pl.BlockSpec((K, bl), lambda i: (0, i)),
            pl.BlockSpec((K + 1, bl), lambda i: (0, i)),
        ],
        out_specs=pl.BlockSpec((K + 1, 8, bl), lambda i: (0, 0, i)),
        out_shape=jax.ShapeDtypeStruct((K + 1, 8, rp), jnp.float32),
    )(uwt, uht)
    return tab_t.reshape((K + 1) * 8, rp).T.reshape(rp * (K + 1), 8)[
        :r * (K + 1)]


def _log_poly(t):
    xi = lax.bitcast_convert_type(t, jnp.int32)
    eb = lax.shift_right_arithmetic(xi, 23) - 127
    mi = lax.bitwise_or(lax.bitwise_and(xi, 0x007FFFFF), 0x3F800000)
    mf = lax.bitcast_convert_type(mi, jnp.float32)
    big = mf > SQRT2
    mf = jnp.where(big, mf * 0.5, mf)
    ef = (eb + big.astype(jnp.int32)).astype(jnp.float32)
    rr = (mf - 1.0) / (mf + 1.0)
    s2 = rr * rr
    lm = rr * (2.0 + s2 * (2.0 / 3.0 + s2 * (2.0 / 5.0 + s2 * (2.0 / 7.0
               + s2 * (2.0 / 9.0)))))
    return ef * LN2 + lm


def _make_sc_kernel(n_pad, npw, chunk, rounds):
    mesh = plsc.VectorSubcoreMesh(core_axis_name="c", subcore_axis_name="s")
    info = plsc.get_sparse_core_info()
    nc = info.num_cores

    @functools.partial(
        pl.kernel,
        mesh=mesh,
        compiler_params=pltpu.CompilerParams(
            needs_layout_passes=False, use_tc_tiling_on_sc=False),
        out_type=[
            jax.ShapeDtypeStruct((n_pad,), jnp.float32),
            jax.ShapeDtypeStruct((n_pad,), jnp.float32),
        ],
        scratch_types=[
            pltpu.VMEM((npw,), jnp.float32),
            pltpu.VMEM((npw,), jnp.int32),
            pltpu.VMEM((NB, chunk, 8), jnp.float32),
            pltpu.VMEM((NB, chunk), jnp.int32),
            pltpu.VMEM((NB, chunk, 8), jnp.float32),
            pltpu.VMEM((npw,), jnp.float32),
            pltpu.VMEM((npw,), jnp.float32),
            [pltpu.SemaphoreType.DMA] * NB,
            [pltpu.SemaphoreType.DMA] * NB,
        ],
    )
    def sc_kernel(x_hbm, ix_hbm, tab_hbm, out_hbm, ld_hbm,
                  x_v, ix17_v, key_v, fidx_v, coef_v, out_v, ld_v,
                  sem1, sem2):
        wid = lax.axis_index("s") * nc + lax.axis_index("c")
        base = pl.multiple_of(wid * npw, 8)
        pltpu.sync_copy(x_hbm.at[pl.ds(base, npw)], x_v)
        pltpu.sync_copy(ix_hbm.at[pl.ds(base, npw)], ix17_v)

        def cvt_body(j, carry):
            jo = pl.multiple_of(j * 64, 16)
            for u in range(4):
                sl = pl.ds(jo + u * 16, 16)
                ix17_v[sl] = ix17_v[sl] * (K + 1)
            return carry

        lax.fori_loop(0, npw // 64, cvt_body, 0)

        def issue1(r, k):
            off = pl.multiple_of(r * chunk, chunk)
            pltpu.async_copy(tab_hbm.at[ix17_v.at[pl.ds(off, chunk)]],
                             key_v.at[k], sem1[k])

        def drain1(k):
            pltpu.make_async_copy(
                tab_hbm.at[ix17_v.at[pl.ds(0, chunk)]],
                key_v.at[k], sem1[k]).wait()

        def issue2(k):
            pltpu.async_copy(tab_hbm.at[fidx_v.at[k]],
                             coef_v.at[k], sem2[k])

        def drain2(k):
            pltpu.make_async_copy(
                tab_hbm.at[fidx_v.at[0]],
                coef_v.at[k], sem2[k]).wait()

        def search(r, k):
            off = pl.multiple_of(r * chunk, chunk)
            lv = key_v.at[k]
            for g in range(chunk // 16):
                go = pl.multiple_of(off + g * 16, 16)
                rowid = lax.iota(jnp.int32, 16) + (g * 16)
                xv = x_v[pl.ds(go, 16)]
                xq = (xv * 65535.0).astype(jnp.int32)
                b = jnp.zeros((16,), jnp.int32)
                for s in (8, 4, 2, 1):
                    t = b + s
                    slot = lax.shift_right_logical(t, 1)
                    pk = plsc.load_gather(lv, [rowid, slot])
                    pi = lax.bitcast_convert_type(pk, jnp.int32)
                    vt = jnp.where((t & 1) == 1,
                                   lax.shift_right_logical(pi, 16),
                                   pi & 0xFFFF)
                    b = jnp.where(vt <= xq, t, b)
                ixv = ix17_v[pl.ds(go, 16)]
                fidx_v[k, pl.ds(g * 16, 16)] = ixv + (b + 1)

        def evaluate(r, k):
            off = pl.multiple_of(r * chunk, chunk)
            cv = coef_v.at[k]
            for g in range(chunk // 16):
                go = pl.multiple_of(off + g * 16, 16)
                xv = x_v[pl.ds(go, 16)]
                rowid = lax.iota(jnp.int32, 16) + (g * 16)
                p_b = plsc.load_gather(cv, [rowid, jnp.zeros((16,), jnp.int32)])
                w_b = plsc.load_gather(cv, [rowid, jnp.full((16,), 1, jnp.int32)])
                iw_b = plsc.load_gather(cv, [rowid, jnp.full((16,), 2, jnp.int32)])
                c_b = plsc.load_gather(cv, [rowid, jnp.full((16,), 3, jnp.int32)])
                h_b = plsc.load_gather(cv, [rowid, jnp.full((16,), 4, jnp.int32)])
                dd = plsc.load_gather(cv, [rowid, jnp.full((16,), 5, jnp.int32)])
                alpha = (xv - p_b) * iw_b
                out_v[pl.ds(go, 16)] = (
                    (0.5 * dd * alpha + h_b) * w_b * alpha + c_b)
                ld_v[pl.ds(go, 16)] = _log_poly(alpha * dd + h_b)

        issue1(0, 0)
        issue1(1, 1)
        drain1(0)
        search(0, 0)
        issue2(0)

        def ring_body(rg, carry):
            r0 = rg * NB
            for k in range(NB):
                r = r0 + k

                @pl.when(r + 2 < rounds)
                def _():
                    issue1(r + 2, (k + 2) % NB)

                @pl.when(r + 1 < rounds)
                def _():
                    drain1((k + 1) % NB)
                    search(r + 1, (k + 1) % NB)
                    issue2((k + 1) % NB)

                drain2(k)
                evaluate(r, k)
            return carry

        lax.fori_loop(0, rounds // NB, ring_body, 0)
        pltpu.sync_copy(out_v, out_hbm.at[pl.ds(base, npw)])
        pltpu.sync_copy(ld_v, ld_hbm.at[pl.ds(base, npw)])

    return sc_kernel


def kernel(x, local_region_ix, unnormalized_widths, unnormalized_heights):
    n = x.shape[0]
    info = plsc.get_sparse_core_info()
    nw = info.num_cores * info.num_subcores
    chunk = 128
    rounds = -(-n // (nw * chunk))
    rounds += (-rounds) % NB
    n_pad = nw * chunk * rounds
    npw = chunk * rounds

    xp = jnp.pad(x, (0, n_pad - n))
    ixp = jnp.pad(local_region_ix.astype(jnp.int32), (0, n_pad - n))
    tab = _build_table(unnormalized_widths, unnormalized_heights)
    out, ld = _make_sc_kernel(n_pad, npw, chunk, rounds)(xp, ixp, tab)
    return out[:n], ld[:n]


# speedup vs baseline: 1.2906x; 1.2906x over previous
import functools

import jax
import jax.numpy as jnp
from jax import lax
from jax.experimental import pallas as pl
from jax.experimental.pallas import tpu as pltpu
from jax.experimental.pallas import tpu_sc as plsc

K = 16
NB = 4
LN2 = 0.6931471805599453
SQRT2 = 1.41421356


def _table_body(uwt_ref, uht_ref, t1_ref, t2_ref):
    uw = uwt_ref[...]
    uh = uht_ref[...]
    m = jnp.max(uw, axis=0, keepdims=True)
    e = jnp.exp(uw - m)
    w = e / jnp.sum(e, axis=0, keepdims=True)
    uhe = jnp.exp(uh)
    hlo = uhe[:K, :]
    hext = uhe[1:K + 1, :]
    pair = 0.5 * (hlo + hext) * w
    area = jnp.sum(pair, axis=0, keepdims=True)
    inv_area = 1.0 / area
    trap = pair * inv_area

    def cumsum0(a):
        for s in (1, 2, 4, 8):
            zz = jnp.zeros((s, a.shape[1]), jnp.float32)
            a = a + jnp.concatenate([zz, a[:K - s, :]], axis=0)
        return a

    cdfc = cumsum0(trap)
    locc = cumsum0(w)
    z = jnp.zeros((1, uw.shape[1]), jnp.float32)
    loc0 = jnp.concatenate([z, locc[:K - 1, :]], axis=0)
    cdf0 = jnp.concatenate([z, cdfc[:K - 1, :]], axis=0)
    h0 = hlo * inv_area
    iw = 1.0 / w
    dd = (hext - hlo) * inv_area
    pieces = []
    for b in range(K):
        for p in (loc0, w, iw, cdf0, h0, dd):
            pieces.append(p[b:b + 1])
        pieces.append(z)
        pieces.append(z)
    big = jnp.concatenate(pieces, axis=0)
    t1_ref[...] = jnp.transpose(loc0)
    t2_ref[...] = jnp.transpose(big)


def _build_tables(uw, uh):
    r = uw.shape[0]
    bl = 6400
    rp = -(-r // bl) * bl
    uwt = jnp.pad(uw.T, ((0, 0), (0, rp - r)))
    uht = jnp.pad(uh.T, ((0, 0), (0, rp - r)))
    t1, t2 = pl.pallas_call(
        _table_body,
        grid=(rp // bl,),
        in_specs=[
            pl.BlockSpec((K, bl), lambda i: (0, i)),
            pl.BlockSpec((K + 1, bl), lambda i: (0, i)),
        ],
        out_specs=[
            pl.BlockSpec((bl, K), lambda i: (i, 0)),
            pl.BlockSpec((bl, 8 * K), lambda i: (i, 0)),
        ],
        out_shape=[
            jax.ShapeDtypeStruct((rp, K), jnp.float32),
            jax.ShapeDtypeStruct((rp, 8 * K), jnp.float32),
        ],
    )(uwt, uht)
    return t1[:r], t2.reshape(rp * 8, K)[:r * 8]


def _log_poly(t):
    xi = lax.bitcast_convert_type(t, jnp.int32)
    eb = lax.shift_right_arithmetic(xi, 23) - 127
    mi = lax.bitwise_or(lax.bitwise_and(xi, 0x007FFFFF), 0x3F800000)
    mf = lax.bitcast_convert_type(mi, jnp.float32)
    big = mf > SQRT2
    mf = jnp.where(big, mf * 0.5, mf)
    ef = (eb + big.astype(jnp.int32)).astype(jnp.float32)
    rr = (mf - 1.0) / (mf + 1.0)
    s2 = rr * rr
    lm = rr * (2.0 + s2 * (2.0 / 3.0 + s2 * (2.0 / 5.0 + s2 * (2.0 / 7.0
               + s2 * (2.0 / 9.0)))))
    return ef * LN2 + lm


def _make_sc_kernel(n_pad, npw, chunk, rounds):
    mesh = plsc.VectorSubcoreMesh(core_axis_name="c", subcore_axis_name="s")
    info = plsc.get_sparse_core_info()
    nc = info.num_cores

    @functools.partial(
        pl.kernel,
        mesh=mesh,
        compiler_params=pltpu.CompilerParams(
            needs_layout_passes=False, use_tc_tiling_on_sc=False),
        out_type=[
            jax.ShapeDtypeStruct((n_pad,), jnp.float32),
            jax.ShapeDtypeStruct((n_pad,), jnp.float32),
        ],
        scratch_types=[
            pltpu.VMEM((npw,), jnp.float32),
            pltpu.VMEM((npw,), jnp.int32),
            pltpu.VMEM((NB, chunk, K), jnp.float32),
            pltpu.VMEM((NB, chunk), jnp.int32),
            pltpu.VMEM((NB, chunk), jnp.int32),
            pltpu.VMEM((NB, chunk, K), jnp.float32),
            pltpu.VMEM((npw,), jnp.float32),
            pltpu.VMEM((npw,), jnp.float32),
            [pltpu.SemaphoreType.DMA] * NB,
            [pltpu.SemaphoreType.DMA] * NB,
        ],
    )
    def sc_kernel(x_hbm, ix_hbm, t1_hbm, t2_hbm, out_hbm, ld_hbm,
                  x_v, ix_v, key_v, fidx_v, cbase_v, coef_v, out_v, ld_v,
                  sem1, sem2):
        wid = lax.axis_index("s") * nc + lax.axis_index("c")
        base = pl.multiple_of(wid * npw, 8)
        pltpu.sync_copy(x_hbm.at[pl.ds(base, npw)], x_v)
        pltpu.sync_copy(ix_hbm.at[pl.ds(base, npw)], ix_v)

        def issue1(r, k):
            off = pl.multiple_of(r * chunk, chunk)
            pltpu.async_copy(t1_hbm.at[ix_v.at[pl.ds(off, chunk)]],
                             key_v.at[k], sem1[k])

        def drain1(k):
            pltpu.make_async_copy(
                t1_hbm.at[ix_v.at[pl.ds(0, chunk)]],
                key_v.at[k], sem1[k]).wait()

        def issue2(k):
            pltpu.async_copy(t2_hbm.at[fidx_v.at[k]],
                             coef_v.at[k], sem2[k])

        def drain2(k):
            pltpu.make_async_copy(
                t2_hbm.at[fidx_v.at[0]],
                coef_v.at[k], sem2[k]).wait()

        def search(r, k):
            off = pl.multiple_of(r * chunk, chunk)
            lv = key_v.at[k]
            for g in range(chunk // 16):
                go = pl.multiple_of(off + g * 16, 16)
                rowid = lax.iota(jnp.int32, 16) + (g * 16)
                xv = x_v[pl.ds(go, 16)]
                b = jnp.zeros((16,), jnp.int32)
                for s in (8, 4, 2, 1):
                    t = b + s
                    pt = plsc.load_gather(lv, [rowid, t])
                    b = jnp.where(pt <= xv, t, b)
                ixv = ix_v[pl.ds(go, 16)]
                sl = pl.ds(g * 16, 16)
                fidx_v[k, sl] = ixv * 8 + lax.shift_right_logical(b, 1)
                cbase_v[k, sl] = lax.shift_left(b & 1, 3)

        def evaluate(r, k):
            off = pl.multiple_of(r * chunk, chunk)
            cv = coef_v.at[k]
            for g in range(chunk // 16):
                go = pl.multiple_of(off + g * 16, 16)
                xv = x_v[pl.ds(go, 16)]
                rowid = lax.iota(jnp.int32, 16) + (g * 16)
                qb = cbase_v[k, pl.ds(g * 16, 16)]
                p_b = plsc.load_gather(cv, [rowid, qb])
                w_b = plsc.load_gather(cv, [rowid, qb + 1])
                iw_b = plsc.load_gather(cv, [rowid, qb + 2])
                c_b = plsc.load_gather(cv, [rowid, qb + 3])
                h_b = plsc.load_gather(cv, [rowid, qb + 4])
                dd = plsc.load_gather(cv, [rowid, qb + 5])
                alpha = (xv - p_b) * iw_b
                out_v[pl.ds(go, 16)] = (
                    (0.5 * dd * alpha + h_b) * w_b * alpha + c_b)
                ld_v[pl.ds(go, 16)] = _log_poly(alpha * dd + h_b)

        issue1(0, 0)
        issue1(1, 1)
        drain1(0)
        search(0, 0)
        issue2(0)

        def ring_body(rg, carry):
            r0 = rg * NB
            for k in range(NB):
                r = r0 + k

                @pl.when(r + 2 < rounds)
                def _():
                    issue1(r + 2, (k + 2) % NB)

                @pl.when(r + 1 < rounds)
                def _():
                    drain1((k + 1) % NB)
                    search(r + 1, (k + 1) % NB)
                    issue2((k + 1) % NB)

                drain2(k)
                evaluate(r, k)
            return carry

        lax.fori_loop(0, rounds // NB, ring_body, 0)
        pltpu.sync_copy(out_v, out_hbm.at[pl.ds(base, npw)])
        pltpu.sync_copy(ld_v, ld_hbm.at[pl.ds(base, npw)])

    return sc_kernel


def kernel(x, local_region_ix, unnormalized_widths, unnormalized_heights):
    n = x.shape[0]
    info = plsc.get_sparse_core_info()
    nw = info.num_cores * info.num_subcores
    chunk = 128
    rounds = -(-n // (nw * chunk))
    rounds += (-rounds) % NB
    n_pad = nw * chunk * rounds
    npw = chunk * rounds

    xp = jnp.pad(x, (0, n_pad - n))
    ixp = jnp.pad(local_region_ix.astype(jnp.int32), (0, n_pad - n))
    t1, t2 = _build_tables(unnormalized_widths, unnormalized_heights)
    out, ld = _make_sc_kernel(n_pad, npw, chunk, rounds)(xp, ixp, t1, t2)
    return out[:n], ld[:n]


# speedup vs baseline: 1.5062x; 1.1670x over previous
import functools

import jax
import jax.numpy as jnp
from jax import lax
from jax.experimental import pallas as pl
from jax.experimental.pallas import tpu as pltpu
from jax.experimental.pallas import tpu_sc as plsc

K = 16
NB = 4
LN2 = 0.6931471805599453
SQRT2 = 1.41421356


def _table_body(uwt_ref, uht_ref, t1_ref, t2_ref):
    uw = uwt_ref[...]
    uh = uht_ref[...]
    m = jnp.max(uw, axis=0, keepdims=True)
    e = jnp.exp(uw - m)
    w = e / jnp.sum(e, axis=0, keepdims=True)
    uhe = jnp.exp(uh)
    hlo = uhe[:K, :]
    hext = uhe[1:K + 1, :]
    pair = 0.5 * (hlo + hext) * w
    area = jnp.sum(pair, axis=0, keepdims=True)
    inv_area = 1.0 / area
    trap = pair * inv_area

    def cumsum0(a):
        for s in (1, 2, 4, 8):
            zz = jnp.zeros((s, a.shape[1]), jnp.float32)
            a = a + jnp.concatenate([zz, a[:K - s, :]], axis=0)
        return a

    cdfc = cumsum0(trap)
    locc = cumsum0(w)
    z = jnp.zeros((1, uw.shape[1]), jnp.float32)
    loc0 = jnp.concatenate([z, locc[:K - 1, :]], axis=0)
    cdf0 = jnp.concatenate([z, cdfc[:K - 1, :]], axis=0)
    h0 = hlo * inv_area
    iw = 1.0 / w
    dd = (hext - hlo) * inv_area
    pieces = []
    for b in range(K):
        for p in (loc0, w, iw, cdf0, h0, dd):
            pieces.append(p[b:b + 1])
        pieces.append(z)
        pieces.append(z)
    big = jnp.concatenate(pieces, axis=0)
    t1_ref[...] = jnp.transpose(loc0)
    t2_ref[...] = jnp.transpose(big)


def _build_tables(uw, uh):
    r = uw.shape[0]
    bl = 6400
    rp = -(-r // bl) * bl
    uwt = jnp.pad(uw.T, ((0, 0), (0, rp - r)))
    uht = jnp.pad(uh.T, ((0, 0), (0, rp - r)))
    t1, t2 = pl.pallas_call(
        _table_body,
        grid=(rp // bl,),
        in_specs=[
            pl.BlockSpec((K, bl), lambda i: (0, i)),
            pl.BlockSpec((K + 1, bl), lambda i: (0, i)),
        ],
        out_specs=[
            pl.BlockSpec((bl, K), lambda i: (i, 0)),
            pl.BlockSpec((bl, 8 * K), lambda i: (i, 0)),
        ],
        out_shape=[
            jax.ShapeDtypeStruct((rp, K), jnp.float32),
            jax.ShapeDtypeStruct((rp, 8 * K), jnp.float32),
        ],
    )(uwt, uht)
    return t1, t2.reshape(rp * 8, K)


def _log_poly(t):
    xi = lax.bitcast_convert_type(t, jnp.int32)
    eb = lax.shift_right_arithmetic(xi, 23) - 127
    mi = lax.bitwise_or(lax.bitwise_and(xi, 0x007FFFFF), 0x3F800000)
    mf = lax.bitcast_convert_type(mi, jnp.float32)
    big = mf > SQRT2
    mf = jnp.where(big, mf * 0.5, mf)
    ef = (eb + big.astype(jnp.int32)).astype(jnp.float32)
    rr = (mf - 1.0) / (mf + 1.0)
    s2 = rr * rr
    lm = rr * (2.0 + s2 * (2.0 / 3.0 + s2 * (2.0 / 5.0 + s2 * (2.0 / 7.0
               + s2 * (2.0 / 9.0)))))
    return ef * LN2 + lm


def _make_sc_kernel(n_pad, npw, chunk, rounds):
    mesh = plsc.VectorSubcoreMesh(core_axis_name="c", subcore_axis_name="s")
    info = plsc.get_sparse_core_info()
    nc = info.num_cores

    @functools.partial(
        pl.kernel,
        mesh=mesh,
        compiler_params=pltpu.CompilerParams(
            needs_layout_passes=False, use_tc_tiling_on_sc=False),
        out_type=[
            jax.ShapeDtypeStruct((n_pad,), jnp.float32),
            jax.ShapeDtypeStruct((n_pad,), jnp.float32),
        ],
        scratch_types=[
            pltpu.VMEM((npw,), jnp.float32),
            pltpu.VMEM((npw,), jnp.int32),
            pltpu.VMEM((NB, chunk, K), jnp.float32),
            pltpu.VMEM((NB, chunk), jnp.int32),
            pltpu.VMEM((NB, chunk), jnp.int32),
            pltpu.VMEM((NB, chunk, K), jnp.float32),
            pltpu.VMEM((npw,), jnp.float32),
            pltpu.VMEM((npw,), jnp.float32),
            [pltpu.SemaphoreType.DMA] * NB,
            [pltpu.SemaphoreType.DMA] * NB,
        ],
    )
    def sc_kernel(x_hbm, ix_hbm, t1_hbm, t2_hbm, out_hbm, ld_hbm,
                  x_v, ix_v, key_v, fidx_v, cbase_v, coef_v, out_v, ld_v,
                  sem1, sem2):
        wid = lax.axis_index("s") * nc + lax.axis_index("c")
        base = pl.multiple_of(wid * npw, 8)
        pltpu.sync_copy(x_hbm.at[pl.ds(base, npw)], x_v)
        pltpu.sync_copy(ix_hbm.at[pl.ds(base, npw)], ix_v)

        def issue1(r, k):
            off = pl.multiple_of(r * chunk, chunk)
            pltpu.async_copy(t1_hbm.at[ix_v.at[pl.ds(off, chunk)]],
                             key_v.at[k], sem1[k])

        def drain1(k):
            pltpu.make_async_copy(
                t1_hbm.at[ix_v.at[pl.ds(0, chunk)]],
                key_v.at[k], sem1[k]).wait()

        def issue2(k):
            pltpu.async_copy(t2_hbm.at[fidx_v.at[k]],
                             coef_v.at[k], sem2[k])

        def drain2(k):
            pltpu.make_async_copy(
                t2_hbm.at[fidx_v.at[0]],
                coef_v.at[k], sem2[k]).wait()

        def search(r, k):
            off = pl.multiple_of(r * chunk, chunk)
            lv = key_v.at[k]
            for g in range(chunk // 16):
                go = pl.multiple_of(off + g * 16, 16)
                rowid = lax.iota(jnp.int32, 16) + (g * 16)
                xv = x_v[pl.ds(go, 16)]
                b = jnp.zeros((16,), jnp.int32)
                for s in (8, 4, 2, 1):
                    t = b + s
                    pt = plsc.load_gather(lv, [rowid, t])
                    b = jnp.where(pt <= xv, t, b)
                ixv = ix_v[pl.ds(go, 16)]
                sl = pl.ds(g * 16, 16)
                fidx_v[k, sl] = ixv * 8 + lax.shift_right_logical(b, 1)
                cbase_v[k, sl] = lax.shift_left(b & 1, 3)

        def evaluate(r, k):
            off = pl.multiple_of(r * chunk, chunk)
            cv = coef_v.at[k]
            for g in range(chunk // 16):
                go = pl.multiple_of(off + g * 16, 16)
                xv = x_v[pl.ds(go, 16)]
                rowid = lax.iota(jnp.int32, 16) + (g * 16)
                qb = cbase_v[k, pl.ds(g * 16, 16)]
                p_b = plsc.load_gather(cv, [rowid, qb])
                w_b = plsc.load_gather(cv, [rowid, qb + 1])
                iw_b = plsc.load_gather(cv, [rowid, qb + 2])
                c_b = plsc.load_gather(cv, [rowid, qb + 3])
                h_b = plsc.load_gather(cv, [rowid, qb + 4])
                dd = plsc.load_gather(cv, [rowid, qb + 5])
                alpha = (xv - p_b) * iw_b
                out_v[pl.ds(go, 16)] = (
                    (0.5 * dd * alpha + h_b) * w_b * alpha + c_b)
                ld_v[pl.ds(go, 16)] = _log_poly(alpha * dd + h_b)

        issue1(0, 0)
        issue1(1, 1)
        drain1(0)
        search(0, 0)
        issue2(0)

        def ring_body(rg, carry):
            r0 = rg * NB
            for k in range(NB):
                r = r0 + k

                @pl.when(r + 2 < rounds)
                def _():
                    issue1(r + 2, (k + 2) % NB)

                @pl.when(r + 1 < rounds)
                def _():
                    drain1((k + 1) % NB)
                    search(r + 1, (k + 1) % NB)
                    issue2((k + 1) % NB)

                drain2(k)
                evaluate(r, k)
            return carry

        lax.fori_loop(0, rounds // NB, ring_body, 0)
        pltpu.sync_copy(out_v, out_hbm.at[pl.ds(base, npw)])
        pltpu.sync_copy(ld_v, ld_hbm.at[pl.ds(base, npw)])

    return sc_kernel


def kernel(x, local_region_ix, unnormalized_widths, unnormalized_heights):
    n = x.shape[0]
    info = plsc.get_sparse_core_info()
    nw = info.num_cores * info.num_subcores
    chunk = 128
    rounds = -(-n // (nw * chunk))
    rounds += (-rounds) % NB
    n_pad = nw * chunk * rounds
    npw = chunk * rounds

    xp = jnp.pad(x, (0, n_pad - n))
    ixp = jnp.pad(local_region_ix.astype(jnp.int32), (0, n_pad - n))
    t1, t2 = _build_tables(unnormalized_widths, unnormalized_heights)
    out, ld = _make_sc_kernel(n_pad, npw, chunk, rounds)(xp, ixp, t1, t2)
    return out[:n], ld[:n]
